# Initial kernel scaffold; baseline (speedup 1.0000x reference)
import jax
import jax.numpy as jnp
from jax.experimental import pallas as pl


def kernel(x, enc_w1, enc_b1, enc_w2, enc_b2, embedding, dec_w1, dec_b1, dec_w2, dec_b2, dec_w3, dec_b3):
    raise NotImplementedError("write your pallas kernel here")


# speedup vs baseline: 1.9087x; 1.9087x over previous
import jax
import jax.numpy as jnp
from jax.experimental import pallas as pl
from jax.experimental.pallas import tpu as pltpu

H = 384
NE = 1024
T = 2048
B = 8
BETA = 0.25


def _dot(a, b):
    return jax.lax.dot_general(a, b, (((1,), (0,)), ((), ())),
                               preferred_element_type=jnp.float32)


def _conv3(v, w3, bcol):
    zero = jnp.zeros((v.shape[0], 1), v.dtype)
    xm = jnp.concatenate([zero, v[:, :-1]], axis=1)
    xp = jnp.concatenate([v[:, 1:], zero], axis=1)
    return _dot(w3[0], xm) + _dot(w3[1], v) + _dot(w3[2], xp) + bcol


def _vqvae_kernel(x_ref, ew1, eb1, ew2, eb2, emb, dw1, db1, dw2, db2, dw3,
                  db3, loss_out, perp_out, xhat_out, loss_acc, cnt_acc):
    b = pl.program_id(0)

    @pl.when(b == 0)
    def _():
        loss_acc[0] = 0.0
        cnt_acc[...] = jnp.zeros_like(cnt_acc)

    xb = x_ref[0]
    h = jnp.maximum(_conv3(xb, ew1[...], eb1[...]), 0.0)
    z = _conv3(h, ew2[...], eb2[...])

    e = emb[...]
    s = _dot(e, z)
    e2 = jnp.sum(e * e, axis=1, keepdims=True)
    c = e2 - 2.0 * s
    z2 = jnp.sum(z * z, axis=0, keepdims=True)
    cmin = jnp.min(c, axis=0, keepdims=True)
    loss_acc[0] += jnp.sum(cmin + z2)
    idx = jnp.argmin(c, axis=0)
    iota = jax.lax.broadcasted_iota(jnp.int32, (NE, T), 0)
    onehot = (iota == idx[None, :]).astype(jnp.float32)
    cnt_acc[...] += jnp.sum(onehot, axis=1, keepdims=True)

    h = jnp.maximum(_conv3(z, dw1[...], db1[...]), 0.0)
    h = jnp.maximum(_conv3(h, dw2[...], db2[...]), 0.0)
    xhat_out[0] = _conv3(h, dw3[...], db3[...])

    @pl.when(b == B - 1)
    def _():
        loss = (1.0 + BETA) * loss_acc[0] / (B * T * H)
        loss_out[...] = jnp.full((1, 1), loss, jnp.float32)
        em = cnt_acc[...] / (B * T)
        ent = jnp.sum(em * jnp.log(em + 1e-10))
        perp_out[...] = jnp.full((1, 1), jnp.exp(-ent), jnp.float32)


def kernel(x, enc_w1, enc_b1, enc_w2, enc_b2, embedding,
           dec_w1, dec_b1, dec_w2, dec_b2, dec_w3, dec_b3):
    tw = lambda w: jnp.transpose(w, (2, 0, 1))
    col = lambda bv: bv.reshape(H, 1)

    full = lambda shp: pl.BlockSpec(shp, lambda b: (0,) * len(shp))
    wspec = full((3, H, H))
    bspec = full((H, 1))

    loss, perp, x_hat = pl.pallas_call(
        _vqvae_kernel,
        grid=(B,),
        in_specs=[
            pl.BlockSpec((1, H, T), lambda b: (b, 0, 0)),
            wspec, bspec, wspec, bspec,
            full((NE, H)),
            wspec, bspec, wspec, bspec, wspec, bspec,
        ],
        out_specs=[
            pl.BlockSpec((1, 1), lambda b: (0, 0)),
            pl.BlockSpec((1, 1), lambda b: (0, 0)),
            pl.BlockSpec((1, H, T), lambda b: (b, 0, 0)),
        ],
        out_shape=[
            jax.ShapeDtypeStruct((1, 1), jnp.float32),
            jax.ShapeDtypeStruct((1, 1), jnp.float32),
            jax.ShapeDtypeStruct((B, H, T), jnp.float32),
        ],
        scratch_shapes=[
            pltpu.SMEM((1,), jnp.float32),
            pltpu.VMEM((NE, 1), jnp.float32),
        ],
    )(x, tw(enc_w1), col(enc_b1), tw(enc_w2), col(enc_b2), embedding,
      tw(dec_w1), col(dec_b1), tw(dec_w2), col(dec_b2), tw(dec_w3),
      col(dec_b3))

    return (loss[0, 0], x_hat, perp[0, 0])
